# baseline (speedup 1.0000x reference)
import jax
import jax.numpy as jnp
from jax import lax
from jax.experimental import pallas as pl
from jax.experimental.pallas import tpu as pltpu
from jax.experimental.pallas import tpu_sc as plsc

_VOCAB = 100
_HIDDEN = 128
_LANES = 128

_NC = 2
_NS = 16
_NW = _NC * _NS

_NB = 5
_IB = 80
_PF = 3


def _gather_body(idx_hbm, tbl_hbm, out_hbm, idxb, rows, tbl_sh,
                 isem, g0, g1, g2, g3, g4, o0, o1, o2, o3, o4):
    n_rows = idx_hbm.shape[0]
    per_w = n_rows // _NW
    n_blocks = per_w // _IB
    wid = lax.axis_index("s") * _NC + lax.axis_index("c")
    base = wid * per_w
    gsem = (g0, g1, g2, g3, g4)
    osem = (o0, o1, o2, o3, o4)

    @pl.when(lax.axis_index("s") == 0)
    def _():
        pltpu.sync_copy(tbl_hbm, tbl_sh)

    plsc.subcore_barrier()

    def blk_load_start(m):
        pltpu.async_copy(
            idx_hbm.at[pl.ds(base + m * _IB, _IB)], idxb.at[lax.rem(m, 2)], isem
        )

    def blk_load_wait(m):
        pltpu.make_async_copy(
            idx_hbm.at[pl.ds(base + m * _IB, _IB)], idxb.at[lax.rem(m, 2)], isem
        ).wait()

    def idx_ref(u):
        return idxb.at[lax.rem(u // _IB, 2), lax.rem(u, _IB)]

    def gather_start(u, b):
        pltpu.async_copy(tbl_sh.at[idx_ref(u)], rows.at[b], gsem[b])

    def gather_wait(u, b):
        pltpu.make_async_copy(tbl_sh.at[idx_ref(u)], rows.at[b], gsem[b]).wait()

    def out_start(u, b):
        pltpu.async_copy(rows.at[b], out_hbm.at[base + u], osem[b])

    def out_wait(u, b):
        pltpu.make_async_copy(rows.at[b], out_hbm.at[base + u], osem[b]).wait()

    blk_load_start(0)
    blk_load_wait(0)
    blk_load_start(1)
    for u in range(_PF):
        gather_start(u, u)

    def step(g, carry):
        for b in range(_NB):
            u = g * _NB + b
            gather_wait(u, b)
            out_start(u, b)

            @pl.when(u + _PF < per_w)
            def _():
                v = u + _PF
                bv = (b + _PF) % _NB

                @pl.when(u >= _NB - _PF)
                def _():
                    out_wait(v - _NB, bv)

                @pl.when(lax.rem(v, _IB) == 0)
                def _():
                    blk_load_wait(v // _IB)

                gather_start(v, bv)

            @pl.when((lax.rem(u, _IB) == _IB - 1) & (u < (n_blocks - 2) * _IB))
            def _():
                blk_load_start((u + 1) // _IB + 1)

        return carry

    lax.fori_loop(0, per_w // _NB, step, 0)

    for i in range(_NB):
        u = per_w - _NB + i
        out_wait(u, u % _NB)


@jax.jit
def _lookup(idx2d, table):
    n_rows = idx2d.shape[0]
    mesh = plsc.VectorSubcoreMesh(core_axis_name="c", subcore_axis_name="s")
    return pl.kernel(
        _gather_body,
        mesh=mesh,
        out_type=jax.ShapeDtypeStruct((n_rows, _LANES, _HIDDEN), jnp.float32),
        scratch_types=[
            pltpu.VMEM((2, _IB, _LANES), jnp.int32),
            pltpu.VMEM((_NB, _LANES, _HIDDEN), jnp.float32),
            pltpu.VMEM_SHARED((_VOCAB, _HIDDEN), jnp.float32),
        ] + [pltpu.SemaphoreType.DMA] * 11,
    )(idx2d, table)


def kernel(indices, word_embeddings):
    b, t = indices.shape
    flat = b * t
    idx2d = indices.reshape(flat // _LANES, _LANES).astype(jnp.int32)
    out = _lookup(idx2d, word_embeddings)
    return out.reshape(b, t, _HIDDEN)
